# baseline (speedup 1.0000x reference)
import dataclasses
import functools

import jax
import jax.numpy as jnp
from jax import lax
from jax.experimental import pallas as pl
from jax.experimental.pallas import tpu as pltpu
from jax.experimental.pallas import tpu_sc as plsc

N_NODES = 10000
N_EDGES = 160000
D_FEAT = 256
DH = 128
NC = 2
NS = 16
C = 40
NITER = 256
E_PAD = NS * NITER * C
BLK = 8
NBLK = NITER // BLK
NB = 8
DEPTH = 3
NI = 3
N_PAD = 10112
ROWS_PER_SUB = N_PAD // NS
CHUNKS = [40] * 15 + [32]


def _gcn_sc_body(x2_hbm, srcb_hbm, dst_hbm, val_hbm, out_hbm,
                 sv, dv, vv, b0, b1, b2, b3, b4, b5, b6, b7, acc_sh,
                 sem_si, sem_di, sem_vi, sem_g, sem_s):
    bufs = (b0, b1, b2, b3, b4, b5, b6, b7)
    c = lax.axis_index("c")
    s = lax.axis_index("s")

    @pl.loop(0, C)
    def _(r):
        for k in range(DH // 16):
            b0.at[r, pl.ds(k * 16, 16)][...] = jnp.zeros((16,), jnp.float32)

    off = 0
    for ch in CHUNKS:
        pltpu.sync_copy(b0.at[pl.ds(0, ch)],
                        acc_sh.at[pl.ds(s * ROWS_PER_SUB + off, ch)])
        off += ch

    plsc.subcore_barrier()

    def idx_descr(g, slot):
        j0 = pl.multiple_of(g * BLK, BLK)
        return (
            pltpu.make_async_copy(
                srcb_hbm.at[c, s, pl.ds(j0, BLK)], sv.at[slot], sem_si.at[slot]),
            pltpu.make_async_copy(
                dst_hbm.at[s, pl.ds(j0, BLK)], dv.at[slot], sem_di.at[slot]),
            pltpu.make_async_copy(
                val_hbm.at[s, pl.ds(j0, BLK)], vv.at[slot], sem_vi.at[slot]),
        )

    def gather_descr(slot, b, q):
        return pltpu.make_async_copy(
            x2_hbm.at[sv.at[slot, b]], bufs[q], sem_g.at[q])

    def scatter_descr(slot, b, q):
        return pltpu.make_async_copy(
            bufs[q], acc_sh.at[dv.at[slot, b]], sem_s.at[q])

    def scale_chunk(slot, b, q):
        p16 = jnp.full((16,), slot, jnp.int32)
        b16 = jnp.full((16,), b, jnp.int32)
        bq = bufs[q]

        @plsc.parallel_loop(0, C, unroll=1)
        def _(e):
            e16 = jnp.full((16,), e, jnp.int32)
            v16 = plsc.load_gather(vv, [p16, b16, e16])
            for k in range(DH // 16):
                sl = pl.ds(k * 16, 16)
                bq.at[e, sl][...] = bq.at[e, sl][...] * v16

    def do_block(g, slot, nslot, first, last):
        if not last:
            for d in idx_descr(g + 1, nslot):
                d.start()
        for b in range(BLK):
            q = b
            q3 = (b + DEPTH) % NB
            if not (first and b < BLK - DEPTH):
                scatter_descr(slot, b, q3).wait()
            if b < BLK - DEPTH:
                gather_descr(slot, b + DEPTH, q3).start()
            elif not last:
                if b == BLK - DEPTH:
                    for d in idx_descr(g + 1, nslot):
                        d.wait()
                gather_descr(nslot, b + DEPTH - BLK, q3).start()
            gather_descr(slot, b, q).wait()
            scale_chunk(slot, b, q)
            pltpu.async_copy(
                bufs[q], acc_sh.at[dv.at[slot, b]], sem_s.at[q], add=True)

    for d in idx_descr(0, 0):
        d.start()
        d.wait()
    for bb in range(DEPTH):
        gather_descr(0, bb, bb).start()

    do_block(0, 0, 1, first=True, last=False)

    @pl.loop(1, NBLK - 4, step=NI)
    def _(g):
        do_block(g, 1, 2, first=False, last=False)
        do_block(g + 1, 2, 0, first=False, last=False)
        do_block(g + 2, 0, 1, first=False, last=False)

    do_block(NBLK - 4, 1, 2, first=False, last=False)
    do_block(NBLK - 3, 2, 0, first=False, last=False)
    do_block(NBLK - 2, 0, 1, first=False, last=False)
    do_block(NBLK - 1, 1, 2, first=False, last=True)

    for bb in range(DEPTH, BLK):
        scatter_descr(1, bb, bb).wait()

    plsc.subcore_barrier()

    off = 0
    for ch in CHUNKS:
        r0 = s * ROWS_PER_SUB + off
        pltpu.sync_copy(acc_sh.at[pl.ds(r0, ch)], out_hbm.at[c, pl.ds(r0, ch)])
        off += ch


@jax.jit
def _gcn_sc(x2, srcb, dst2, val2):
    mesh = plsc.VectorSubcoreMesh(core_axis_name="c", subcore_axis_name="s")
    cp = pltpu.CompilerParams()
    if "needs_layout_passes" in pltpu.CompilerParams.__dataclass_fields__:
        cp = dataclasses.replace(cp, needs_layout_passes=False)
    kern = functools.partial(
        pl.kernel,
        mesh=mesh,
        compiler_params=cp,
        out_type=jax.ShapeDtypeStruct((NC, N_PAD, DH), jnp.float32),
        scratch_types=[
            pltpu.VMEM((NI, BLK, C), jnp.int32),
            pltpu.VMEM((NI, BLK, C), jnp.int32),
            pltpu.VMEM((NI, BLK, C), jnp.float32),
        ] + [pltpu.VMEM((C, DH), jnp.float32)] * NB + [
            pltpu.VMEM_SHARED((N_PAD, DH), jnp.float32),
            pltpu.SemaphoreType.DMA((NI,)),
            pltpu.SemaphoreType.DMA((NI,)),
            pltpu.SemaphoreType.DMA((NI,)),
            pltpu.SemaphoreType.DMA((NB,)),
            pltpu.SemaphoreType.DMA((NB,)),
        ],
    )(_gcn_sc_body)
    return kern(x2, srcb, dst2, val2)


def kernel(x, edge_index, adj_values):
    src = edge_index[0].astype(jnp.int32)
    dst = edge_index[1].astype(jnp.int32)
    vals = adj_values.astype(jnp.float32)
    pad = E_PAD - N_EDGES
    src = jnp.concatenate([src, jnp.zeros((pad,), jnp.int32)])
    dst = jnp.concatenate([dst, jnp.zeros((pad,), jnp.int32)])
    vals = jnp.concatenate([vals, jnp.zeros((pad,), jnp.float32)])
    x2 = jnp.concatenate([x[:, :DH], x[:, DH:]], axis=0)
    srcb = jnp.stack([src, src + N_NODES]).reshape(NC, NS, NITER, C)
    dst2 = dst.reshape(NS, NITER, C)
    val2 = vals.reshape(NS, NITER, C)
    out2 = _gcn_sc(x2, srcb, dst2, val2)
    return jnp.concatenate([out2[0, :N_NODES], out2[1, :N_NODES]], axis=1)


# speedup vs baseline: 2.0419x; 2.0419x over previous
import dataclasses
import functools

import jax
import jax.numpy as jnp
from jax import lax
from jax.experimental import pallas as pl
from jax.experimental.pallas import tpu as pltpu
from jax.experimental.pallas import tpu_sc as plsc

N_NODES = 10000
N_EDGES = 160000
D_FEAT = 256
DH = 128
NC = 2
NS = 16
C = 50
NITER = 200
E_PAD = NS * NITER * C
BLK = 8
NBLK = NITER // BLK
NB = 4
DEPTH = 3
NI = 3
N_PAD = 10112
ROWS_PER_SUB = N_PAD // NS
CHUNKS = [48] * 13 + [8]


def _gcn_sc_body(x2_hbm, srcb_hbm, dst_hbm, val_hbm, out_hbm,
                 sv, dv, vv, b0, b1, b2, b3, acc_sh,
                 sem_si, sem_di, sem_vi, sem_g, sem_s):
    bufs = (b0, b1, b2, b3)
    c = lax.axis_index("c")
    s = lax.axis_index("s")

    @pl.loop(0, C)
    def _(r):
        for k in range(DH // 16):
            b0.at[r, pl.ds(k * 16, 16)][...] = jnp.zeros((16,), jnp.float32)

    off = 0
    for ch in CHUNKS:
        pltpu.sync_copy(b0.at[pl.ds(0, ch)],
                        acc_sh.at[pl.ds(s * ROWS_PER_SUB + off, ch)])
        off += ch

    plsc.subcore_barrier()

    def idx_descr(g, slot):
        j0 = pl.multiple_of(g * BLK, BLK)
        return (
            pltpu.make_async_copy(
                srcb_hbm.at[c, s, pl.ds(j0, BLK)], sv.at[slot], sem_si.at[slot]),
            pltpu.make_async_copy(
                dst_hbm.at[s, pl.ds(j0, BLK)], dv.at[slot], sem_di.at[slot]),
            pltpu.make_async_copy(
                val_hbm.at[s, pl.ds(j0, BLK)], vv.at[slot], sem_vi.at[slot]),
        )

    def gather_descr(slot, b, q):
        return pltpu.make_async_copy(
            x2_hbm.at[sv.at[slot, b]], bufs[q], sem_g.at[q])

    def scatter_descr(slot, b, q):
        return pltpu.make_async_copy(
            bufs[q], acc_sh.at[dv.at[slot, b]], sem_s.at[q])

    def scale_chunk(slot, b, q):
        p16 = jnp.full((16,), slot, jnp.int32)
        b16 = jnp.full((16,), b, jnp.int32)
        bq = bufs[q]

        @plsc.parallel_loop(0, C, unroll=1)
        def _(e):
            e16 = jnp.full((16,), e, jnp.int32)
            v16 = plsc.load_gather(vv, [p16, b16, e16])
            for k in range(DH // 16):
                sl = pl.ds(k * 16, 16)
                bq.at[e, sl][...] = bq.at[e, sl][...] * v16

    def do_block(g, slot, nslot, first, last):
        if not last:
            for d in idx_descr(g + 1, nslot):
                d.start()
        for b in range(BLK):
            q = b % NB
            q3 = (b + DEPTH) % NB
            if not (first and b < BLK - DEPTH):
                scatter_descr(slot, b, q3).wait()
            if b < BLK - DEPTH:
                gather_descr(slot, b + DEPTH, q3).start()
            elif not last:
                if b == BLK - DEPTH:
                    for d in idx_descr(g + 1, nslot):
                        d.wait()
                gather_descr(nslot, b + DEPTH - BLK, q3).start()
            gather_descr(slot, b, q).wait()
            scale_chunk(slot, b, q)
            pltpu.async_copy(
                bufs[q], acc_sh.at[dv.at[slot, b]], sem_s.at[q], add=True)

    for d in idx_descr(0, 0):
        d.start()
        d.wait()
    for bb in range(DEPTH):
        gather_descr(0, bb, bb).start()

    do_block(0, 0, 1, first=True, last=False)

    @pl.loop(1, NBLK - 3, step=NI)
    def _(g):
        do_block(g, 1, 2, first=False, last=False)
        do_block(g + 1, 2, 0, first=False, last=False)
        do_block(g + 2, 0, 1, first=False, last=False)

    do_block(NBLK - 3, 1, 2, first=False, last=False)
    do_block(NBLK - 2, 2, 0, first=False, last=False)
    do_block(NBLK - 1, 0, 1, first=False, last=True)

    for bb in range(DEPTH, BLK):
        scatter_descr(0, bb, bb % NB).wait()

    plsc.subcore_barrier()

    off = 0
    for ch in CHUNKS:
        r0 = s * ROWS_PER_SUB + off
        pltpu.sync_copy(acc_sh.at[pl.ds(r0, ch)], out_hbm.at[c, pl.ds(r0, ch)])
        off += ch


@jax.jit
def _gcn_sc(x2, srcb, dst2, val2):
    mesh = plsc.VectorSubcoreMesh(core_axis_name="c", subcore_axis_name="s")
    cp = pltpu.CompilerParams()
    if "needs_layout_passes" in pltpu.CompilerParams.__dataclass_fields__:
        cp = dataclasses.replace(cp, needs_layout_passes=False)
    kern = functools.partial(
        pl.kernel,
        mesh=mesh,
        compiler_params=cp,
        out_type=jax.ShapeDtypeStruct((NC, N_PAD, DH), jnp.float32),
        scratch_types=[
            pltpu.VMEM((NI, BLK, C), jnp.int32),
            pltpu.VMEM((NI, BLK, C), jnp.int32),
            pltpu.VMEM((NI, BLK, C), jnp.float32),
        ] + [pltpu.VMEM((C, DH), jnp.float32)] * NB + [
            pltpu.VMEM_SHARED((N_PAD, DH), jnp.float32),
            pltpu.SemaphoreType.DMA((NI,)),
            pltpu.SemaphoreType.DMA((NI,)),
            pltpu.SemaphoreType.DMA((NI,)),
            pltpu.SemaphoreType.DMA((NB,)),
            pltpu.SemaphoreType.DMA((NB,)),
        ],
    )(_gcn_sc_body)
    return kern(x2, srcb, dst2, val2)


def kernel(x, edge_index, adj_values):
    src = edge_index[0].astype(jnp.int32)
    dst = edge_index[1].astype(jnp.int32)
    vals = adj_values.astype(jnp.float32)
    x2 = jnp.concatenate([x[:, :DH], x[:, DH:]], axis=0)
    srcb = jnp.stack([src, src + N_NODES]).reshape(NC, NS, NITER, C)
    dst2 = dst.reshape(NS, NITER, C)
    val2 = vals.reshape(NS, NITER, C)
    out2 = _gcn_sc(x2, srcb, dst2, val2)
    return jnp.concatenate([out2[0, :N_NODES], out2[1, :N_NODES]], axis=1)
